# baseline (speedup 1.0000x reference)
import jax
import jax.numpy as jnp
from jax.experimental import pallas as pl
from jax.experimental.pallas import tpu as pltpu

V = 32000
B = 4096
LABEL_SMOOTHING = 0.1
CONFIDENCE = 1.0 - LABEL_SMOOTHING
IGNORE_COL = (-100) % V
SMOOTH = LABEL_SMOOTHING / (V - 2)

ROWS_PER_BLOCK = 128
LANES = 128


def _loss_block_kernel(x_ref, t_ref, out_ref):
    i = pl.program_id(0)
    t = t_ref[0, 0, :]
    r = ROWS_PER_BLOCK
    n_chunks = V // LANES

    def pass_a(k, carry):
        macc, tacc = carry
        c = x_ref[:, pl.ds(k * LANES, LANES)]
        col = k * LANES + jax.lax.broadcasted_iota(jnp.int32, (r, LANES), 1)
        macc = jnp.maximum(macc, c)
        tacc = tacc + jnp.where(col == t[:, None], c, 0.0)
        return macc, tacc

    macc, tacc = jax.lax.fori_loop(
        0, n_chunks, pass_a,
        (jnp.full((r, LANES), -jnp.inf, jnp.float32),
         jnp.zeros((r, LANES), jnp.float32)))
    m = jnp.max(macc, axis=1)
    x_t = jnp.sum(tacc, axis=1)

    def pass_b(k, carry):
        eacc, sacc = carry
        c = x_ref[:, pl.ds(k * LANES, LANES)]
        eacc = eacc + jnp.exp(c - m[:, None])
        sacc = sacc + c
        return eacc, sacc

    eacc, sacc = jax.lax.fori_loop(
        0, n_chunks, pass_b,
        (jnp.zeros((r, LANES), jnp.float32),
         jnp.zeros((r, LANES), jnp.float32)))
    se = jnp.sum(eacc, axis=1)
    sx = jnp.sum(sacc, axis=1)
    lse = m + jnp.log(se)

    x_i = x_ref[:, IGNORE_COL]
    logp_t = x_t - lse
    logp_i = x_i - lse
    is_i = (t == IGNORE_COL).astype(jnp.float32)

    slog_s = SMOOTH * jnp.log(SMOOTH)
    clog_c = CONFIDENCE * jnp.log(CONFIDENCE)
    base = (V - 2 + is_i) * slog_s + clog_c
    cross = (SMOOTH * (sx - V * lse)
             + (CONFIDENCE - SMOOTH) * logp_t
             - SMOOTH * logp_i
             + is_i * SMOOTH * logp_i)
    partial = jnp.sum(base - cross)

    @pl.when(i == 0)
    def _init():
        out_ref[0, 0] = 0.0

    out_ref[0, 0] += partial


@jax.jit
def kernel(output, target, one_hot):
    del one_hot
    b, v = output.shape
    r = ROWS_PER_BLOCK
    grid = b // r
    t3 = target.astype(jnp.int32).reshape(grid, 1, r)
    total = pl.pallas_call(
        _loss_block_kernel,
        grid=(grid,),
        in_specs=[
            pl.BlockSpec((r, v), lambda i: (i, 0)),
            pl.BlockSpec((1, 1, r), lambda i: (i, 0, 0)),
        ],
        out_specs=pl.BlockSpec(memory_space=pltpu.SMEM),
        out_shape=jax.ShapeDtypeStruct((1, 1), jnp.float32),
    )(output, t3)
    return (total[0, 0] / b).astype(jnp.float32)


# speedup vs baseline: 1.6334x; 1.6334x over previous
import functools

import jax
import jax.numpy as jnp
from jax import lax
from jax.experimental import pallas as pl
from jax.experimental.pallas import tpu as pltpu
from jax.experimental.pallas import tpu_sc as plsc

V = 32000
B = 4096
LABEL_SMOOTHING = 0.1
CONFIDENCE = 1.0 - LABEL_SMOOTHING
IGNORE_COL = (-100) % V
SMOOTH = LABEL_SMOOTHING / (V - 2)

ROWS_PER_BLOCK = 128

_NC = 2
_NS = 16
_NW = _NC * _NS
_CHUNK = B // _NW
_L = 16

_sc_mesh = plsc.VectorSubcoreMesh(core_axis_name="c", subcore_axis_name="s")


@functools.partial(
    pl.kernel,
    mesh=_sc_mesh,
    out_type=jax.ShapeDtypeStruct((B,), jnp.float32),
    scratch_types=[
        pltpu.VMEM((_CHUNK,), jnp.int32),
        pltpu.VMEM((_CHUNK,), jnp.int32),
        pltpu.VMEM((_CHUNK,), jnp.float32),
        pltpu.SemaphoreType.DMA,
    ],
)
def _gather_xt(flat_hbm, tgt_hbm, out_hbm, t_v, idx_v, val_v, sem):
    wid = lax.axis_index("s") * _NC + lax.axis_index("c")
    base = wid * _CHUNK
    pltpu.sync_copy(tgt_hbm.at[pl.ds(base, _CHUNK)], t_v)
    for j in range(_CHUNK // _L):
        t16 = t_v[pl.ds(j * _L, _L)]
        rows = base + j * _L + lax.iota(jnp.int32, _L)
        idx_v[pl.ds(j * _L, _L)] = rows * V + t16
    pltpu.async_copy(flat_hbm.at[idx_v], val_v, sem).wait()
    pltpu.sync_copy(val_v, out_hbm.at[pl.ds(base, _CHUNK)])


def _loss_block_kernel(x_ref, t_ref, xt_ref, out_ref):
    i = pl.program_id(0)
    x = x_ref[...]
    t = t_ref[0, 0, :]
    x_t = xt_ref[0, 0, :]

    m = jnp.max(x, axis=1, keepdims=True)
    se = jnp.sum(jnp.exp(x - m), axis=1)
    lse = m[:, 0] + jnp.log(se)
    sx = jnp.sum(x, axis=1)
    x_i = x[:, IGNORE_COL]

    logp_t = x_t - lse
    logp_i = x_i - lse
    is_i = (t == IGNORE_COL).astype(jnp.float32)

    slog_s = SMOOTH * jnp.log(SMOOTH)
    clog_c = CONFIDENCE * jnp.log(CONFIDENCE)
    base = (V - 2 + is_i) * slog_s + clog_c
    cross = (SMOOTH * (sx - V * lse)
             + (CONFIDENCE - SMOOTH) * logp_t
             - SMOOTH * logp_i
             + is_i * SMOOTH * logp_i)
    partial = jnp.sum(base - cross)

    @pl.when(i == 0)
    def _init():
        out_ref[0, 0] = 0.0

    out_ref[0, 0] += partial


@jax.jit
def kernel(output, target, one_hot):
    del one_hot
    b, v = output.shape
    r = ROWS_PER_BLOCK
    grid = b // r
    tgt = target.astype(jnp.int32)
    xt = _gather_xt(output.reshape(-1), tgt)
    t3 = tgt.reshape(grid, 1, r)
    xt3 = xt.reshape(grid, 1, r)
    total = pl.pallas_call(
        _loss_block_kernel,
        grid=(grid,),
        in_specs=[
            pl.BlockSpec((r, v), lambda i: (i, 0)),
            pl.BlockSpec((1, 1, r), lambda i: (i, 0, 0)),
            pl.BlockSpec((1, 1, r), lambda i: (i, 0, 0)),
        ],
        out_specs=pl.BlockSpec(memory_space=pltpu.SMEM),
        out_shape=jax.ShapeDtypeStruct((1, 1), jnp.float32),
    )(output, t3, xt3)
    return (total[0, 0] / b).astype(jnp.float32)


# speedup vs baseline: 3.2325x; 1.9790x over previous
import jax
import jax.numpy as jnp
from jax import lax
from jax.experimental import pallas as pl
from jax.experimental.pallas import tpu as pltpu

V = 32000
B = 4096
LABEL_SMOOTHING = 0.1
CONFIDENCE = 1.0 - LABEL_SMOOTHING
IGNORE_COL = (-100) % V
SMOOTH = LABEL_SMOOTHING / (V - 2)

ROWS_PER_BLOCK = 128
LANES = 128


def _loss_block_kernel(hbm_ref, x_ref, t_ref, ts_ref, out_ref, g_ref, sem):
    i = pl.program_id(0)
    r = ROWS_PER_BLOCK
    x = x_ref[...]
    t = t_ref[0, 0, :]

    def issue(rr, carry):
        tr = ts_ref[0, 0, rr]
        chunk = tr // LANES
        pltpu.make_async_copy(
            hbm_ref.at[i * r + rr, pl.ds(chunk * LANES, LANES)],
            g_ref.at[rr], sem).start()
        return carry

    lax.fori_loop(0, r, issue, 0)

    m = jnp.max(x, axis=1, keepdims=True)
    se = jnp.sum(jnp.exp(x - m), axis=1)
    lse = m[:, 0] + jnp.log(se)
    sx = jnp.sum(x, axis=1)
    x_i = x[:, IGNORE_COL]

    def drain(rr, carry):
        pltpu.make_async_copy(
            hbm_ref.at[0, pl.ds(0, LANES)], g_ref.at[rr], sem).wait()
        return carry

    lax.fori_loop(0, r, drain, 0)

    g = g_ref[...]
    lane = t % LANES
    li = lax.broadcasted_iota(jnp.int32, (r, LANES), 1)
    x_t = jnp.sum(jnp.where(li == lane[:, None], g, 0.0), axis=1)

    logp_t = x_t - lse
    logp_i = x_i - lse
    is_i = (t == IGNORE_COL).astype(jnp.float32)

    slog_s = SMOOTH * jnp.log(SMOOTH)
    clog_c = CONFIDENCE * jnp.log(CONFIDENCE)
    base = (V - 2 + is_i) * slog_s + clog_c
    cross = (SMOOTH * (sx - V * lse)
             + (CONFIDENCE - SMOOTH) * logp_t
             - SMOOTH * logp_i
             + is_i * SMOOTH * logp_i)
    partial = jnp.sum(base - cross)

    @pl.when(i == 0)
    def _init():
        out_ref[0, 0] = 0.0

    out_ref[0, 0] += partial


@jax.jit
def kernel(output, target, one_hot):
    del one_hot
    b, v = output.shape
    r = ROWS_PER_BLOCK
    grid = b // r
    t3 = target.astype(jnp.int32).reshape(grid, 1, r)
    total = pl.pallas_call(
        _loss_block_kernel,
        grid=(grid,),
        in_specs=[
            pl.BlockSpec(memory_space=pl.ANY),
            pl.BlockSpec((r, v), lambda i: (i, 0)),
            pl.BlockSpec((1, 1, r), lambda i: (i, 0, 0)),
            pl.BlockSpec((1, 1, r), lambda i: (i, 0, 0),
                         memory_space=pltpu.SMEM),
        ],
        out_specs=pl.BlockSpec(memory_space=pltpu.SMEM),
        out_shape=jax.ShapeDtypeStruct((1, 1), jnp.float32),
        scratch_shapes=[
            pltpu.VMEM((r, LANES), jnp.float32),
            pltpu.SemaphoreType.DMA,
        ],
    )(output, output, t3, t3)
    return (total[0, 0] / b).astype(jnp.float32)


# speedup vs baseline: 3.6136x; 1.1179x over previous
import jax
import jax.numpy as jnp
from jax import lax
from jax.experimental import pallas as pl
from jax.experimental.pallas import tpu as pltpu

V = 32000
B = 4096
LABEL_SMOOTHING = 0.1
CONFIDENCE = 1.0 - LABEL_SMOOTHING
IGNORE_COL = (-100) % V
SMOOTH = LABEL_SMOOTHING / (V - 2)

ROWS_PER_BLOCK = 128
LANES = 128


def _loss_block_kernel(x_ref, t_ref, ones_ref, out_ref):
    i = pl.program_id(0)
    r = ROWS_PER_BLOCK
    x = x_ref[...]
    t = t_ref[0, 0, :]

    m = jnp.max(x, axis=1, keepdims=True)
    se = jnp.sum(jnp.exp(x - m), axis=1)
    lse = m[:, 0] + jnp.log(se)
    sx = lax.dot_general(x, ones_ref[...],
                         (((1,), (0,)), ((), ())),
                         preferred_element_type=jnp.float32)[:, 0]
    x_i = x[:, IGNORE_COL]

    col = lax.broadcasted_iota(jnp.int32, (r, V), 1)
    x_t = jnp.sum(jnp.where(col == t[:, None], x, 0.0), axis=1)

    logp_t = x_t - lse
    logp_i = x_i - lse
    is_i = (t == IGNORE_COL).astype(jnp.float32)

    slog_s = SMOOTH * jnp.log(SMOOTH)
    clog_c = CONFIDENCE * jnp.log(CONFIDENCE)
    base = (V - 2 + is_i) * slog_s + clog_c
    cross = (SMOOTH * (sx - V * lse)
             + (CONFIDENCE - SMOOTH) * logp_t
             - SMOOTH * logp_i
             + is_i * SMOOTH * logp_i)
    partial = jnp.sum(base - cross)

    @pl.when(i == 0)
    def _init():
        out_ref[0, 0] = 0.0

    out_ref[0, 0] += partial


@jax.jit
def kernel(output, target, one_hot):
    del one_hot
    b, v = output.shape
    r = ROWS_PER_BLOCK
    grid = b // r
    t3 = target.astype(jnp.int32).reshape(grid, 1, r)
    ones = jnp.ones((v, LANES), jnp.float32)
    total = pl.pallas_call(
        _loss_block_kernel,
        grid=(grid,),
        in_specs=[
            pl.BlockSpec((r, v), lambda i: (i, 0)),
            pl.BlockSpec((1, 1, r), lambda i: (i, 0, 0)),
            pl.BlockSpec((v, LANES), lambda i: (0, 0)),
        ],
        out_specs=pl.BlockSpec(memory_space=pltpu.SMEM),
        out_shape=jax.ShapeDtypeStruct((1, 1), jnp.float32),
    )(output, t3, ones)
    return (total[0, 0] / b).astype(jnp.float32)


# speedup vs baseline: 4.7076x; 1.3028x over previous
import jax
import jax.numpy as jnp
from jax import lax
from jax.experimental import pallas as pl
from jax.experimental.pallas import tpu as pltpu

V = 32000
B = 4096
LABEL_SMOOTHING = 0.1
CONFIDENCE = 1.0 - LABEL_SMOOTHING
IGNORE_COL = (-100) % V
SMOOTH = LABEL_SMOOTHING / (V - 2)

ROWS_PER_BLOCK = 128


def _loss_block_kernel(x_ref, t_ref, out_ref):
    i = pl.program_id(0)
    r = ROWS_PER_BLOCK
    x = x_ref[...]
    t = t_ref[0, 0, :]

    m = jnp.max(x, axis=1, keepdims=True)
    se = jnp.sum(jnp.exp(x - m), axis=1)
    lse = m[:, 0] + jnp.log(se)

    col = lax.broadcasted_iota(jnp.int32, (r, V), 1)
    w = jnp.where(col == t[:, None], CONFIDENCE, SMOOTH)
    fused = jnp.sum(x * w, axis=1)

    x_i = x[:, IGNORE_COL]
    is_i = (t == IGNORE_COL).astype(jnp.float32)

    slog_s = SMOOTH * jnp.log(SMOOTH)
    clog_c = CONFIDENCE * jnp.log(CONFIDENCE)
    base = (V - 2 + is_i) * slog_s + clog_c
    cross = fused - (1.0 - is_i) * SMOOTH * x_i - lse * (1.0 + is_i * SMOOTH)
    partial = jnp.sum(base - cross)

    @pl.when(i == 0)
    def _init():
        out_ref[0, 0] = 0.0

    out_ref[0, 0] += partial


@jax.jit
def kernel(output, target, one_hot):
    del one_hot
    b, v = output.shape
    r = ROWS_PER_BLOCK
    grid = b // r
    t3 = target.astype(jnp.int32).reshape(grid, 1, r)
    total = pl.pallas_call(
        _loss_block_kernel,
        grid=(grid,),
        in_specs=[
            pl.BlockSpec((r, v), lambda i: (i, 0)),
            pl.BlockSpec((1, 1, r), lambda i: (i, 0, 0)),
        ],
        out_specs=pl.BlockSpec(memory_space=pltpu.SMEM),
        out_shape=jax.ShapeDtypeStruct((1, 1), jnp.float32),
    )(output, t3)
    return (total[0, 0] / b).astype(jnp.float32)
